# Initial kernel scaffold; baseline (speedup 1.0000x reference)
import jax
import jax.numpy as jnp
from jax.experimental import pallas as pl


def kernel(user_id_idx, user_hist_idx, user_hist_mask, item_id_idx, item_cate_idx, user_id_table, item_id_table, item_cate_table):
    raise NotImplementedError("write your pallas kernel here")


# speedup vs baseline: 5.6864x; 5.6864x over previous
import functools

import jax
import jax.numpy as jnp
from jax import lax
from jax.experimental import pallas as pl
from jax.experimental.pallas import tpu as pltpu
from jax.experimental.pallas import tpu_sc as plsc

B = 4096
L = 50
D = 64
NL = 16
NW = 32
BPW = B // NW
CB = 8
NCHUNK = BPW // CB
LP = 64
IDXW = 100
NDMA = CB * L // IDXW
HROWS = BPW * L // IDXW

_mesh = plsc.VectorSubcoreMesh(core_axis_name="c", subcore_axis_name="s")


@functools.partial(
    pl.kernel,
    mesh=_mesh,
    compiler_params=pltpu.CompilerParams(use_tc_tiling_on_sc=False,
                                         needs_layout_passes=False),
    out_type=jax.ShapeDtypeStruct((B, 4 * D), jnp.float32),
    scratch_types=[
        pltpu.VMEM((BPW,), jnp.int32),
        pltpu.VMEM((BPW, D), jnp.float32),
        pltpu.VMEM((HROWS, IDXW), jnp.int32),
        pltpu.VMEM((BPW * LP,), jnp.float32),
        pltpu.VMEM((CB * L, D), jnp.float32),
        pltpu.VMEM((BPW, 4 * D), jnp.float32),
        pltpu.SemaphoreType.DMA,
    ],
)
def _sc_fwd(uid_hbm, hidx_hbm, mask_hbm, iid_hbm, cid_hbm,
            utab_hbm, itab_hbm, ctab_hbm, out_hbm,
            idx_v, rows_v, hidx_v, mask_v, hrows_v, outv, sem):
    wid = lax.axis_index("s") * 2 + lax.axis_index("c")
    base = wid * BPW

    for src_idx, tab, col in ((uid_hbm, utab_hbm, 0),
                              (iid_hbm, itab_hbm, 2),
                              (cid_hbm, ctab_hbm, 3)):
        pltpu.sync_copy(src_idx.at[pl.ds(base, BPW)], idx_v)
        pltpu.async_copy(tab.at[idx_v], rows_v, sem).wait()

        def copy_row(r, carry):
            for k in range(D // NL):
                outv[r, pl.ds(col * D + k * NL, NL)] = rows_v[r, pl.ds(k * NL, NL)]
            return carry

        lax.fori_loop(0, BPW, copy_row, 0)

    pltpu.sync_copy(hidx_hbm.at[pl.ds(wid * HROWS, HROWS)], hidx_v)
    pltpu.sync_copy(mask_hbm.at[pl.ds(base * LP, BPW * LP)], mask_v)

    zero = jnp.zeros((NL,), jnp.float32)

    def chunk_body(c, carry):
        cps = [
            pltpu.async_copy(itab_hbm.at[hidx_v.at[c * NDMA + j]],
                             hrows_v.at[pl.ds(j * IDXW, IDXW)], sem)
            for j in range(NDMA)
        ]
        for cp in cps:
            cp.wait()

        def b_body(bl, carry2):
            row = c * CB + bl
            boff = row * LP
            m = [mask_v[pl.ds(boff + j * NL, NL)] for j in range(LP // NL)]
            den = jnp.broadcast_to(jnp.sum(m[0] + m[1] + m[2] + m[3]), (NL,))
            s = jnp.float32(1.0) / den
            a = [zero, zero, zero, zero]
            for l in range(L):
                w = m[l // NL][l % NL]
                r = bl * L + l
                for k in range(D // NL):
                    a[k] = a[k] + w * hrows_v[r, pl.ds(k * NL, NL)]
            for k in range(D // NL):
                outv[row, pl.ds(D + k * NL, NL)] = a[k] * s
            return carry2

        lax.fori_loop(0, CB, b_body, 0)
        return carry

    lax.fori_loop(0, NCHUNK, chunk_body, 0)
    pltpu.sync_copy(outv, out_hbm.at[pl.ds(base, BPW), :])


def kernel(user_id_idx, user_hist_idx, user_hist_mask, item_id_idx,
           item_cate_idx, user_id_table, item_id_table, item_cate_table):
    uid = user_id_idx.reshape(B).astype(jnp.int32)
    iid = item_id_idx.reshape(B).astype(jnp.int32)
    cid = item_cate_idx.reshape(B).astype(jnp.int32)
    hidx = user_hist_idx.astype(jnp.int32).reshape(B * L // IDXW, IDXW)
    maskf = jnp.pad(user_hist_mask.astype(jnp.float32),
                    ((0, 0), (0, LP - L))).reshape(B * LP)
    return _sc_fwd(uid, hidx, maskf, iid, cid,
                   user_id_table.astype(jnp.float32),
                   item_id_table.astype(jnp.float32),
                   item_cate_table.astype(jnp.float32))
